# baseline (speedup 1.0000x reference)
import functools

import jax
import jax.numpy as jnp
from jax import lax
from jax.experimental import pallas as pl
from jax.experimental.pallas import tpu as pltpu
from jax.experimental.pallas import tpu_sc as plsc

_B = 16384
_DH = 128
_DC = 128
_ND = 8
_EPS = 1e-05

_NW = 32
_RPW = _B // _NW
_CH = 128
_NCH = _RPW // _CH
_LANES = 16

_BB = 4096
_NB = _B // _BB


def _sc_gather_body(ids_hbm, table_hbm, out_hbm, ids_v, idx_v, rows_v, sem):
    wid = lax.axis_index("s") * 2 + lax.axis_index("c")
    base = wid * _RPW
    pltpu.sync_copy(ids_hbm.at[pl.ds(base, _RPW)], ids_v)

    for r in range(_NCH):

        def _mk_idx(j, carry, r=r):
            col = r * _CH + j * _LANES
            rows = base + col + lax.iota(jnp.int32, _LANES)
            idx_v[r, pl.ds(j * _LANES, _LANES)] = (
                ids_v[pl.ds(col, _LANES)] * _B + rows
            )
            return carry

        lax.fori_loop(0, _CH // _LANES, _mk_idx, 0)
    gathers = [
        pltpu.async_copy(
            table_hbm.at[idx_v.at[r]], rows_v.at[pl.ds(r * _CH, _CH)], sem
        )
        for r in range(_NCH)
    ]
    for cp in gathers:
        cp.wait()
    pltpu.sync_copy(rows_v, out_hbm.at[pl.ds(base, _RPW)])


@functools.lru_cache(maxsize=1)
def _sc_gather_call():
    mesh = plsc.VectorSubcoreMesh(core_axis_name="c", subcore_axis_name="s")
    return pl.kernel(
        _sc_gather_body,
        out_type=jax.ShapeDtypeStruct((_B, _DC), jnp.float32),
        mesh=mesh,
        scratch_types=[
            pltpu.VMEM((_RPW,), jnp.int32),
            pltpu.VMEM((_NCH, _CH), jnp.int32),
            pltpu.VMEM((_RPW, _DC), jnp.float32),
            pltpu.SemaphoreType.DMA,
        ],
    )


def _onehot_t(ids_ref):
    ids = jnp.broadcast_to(ids_ref[0], (_ND, _BB))
    return (ids == lax.broadcasted_iota(jnp.int32, (_ND, _BB), 0)).astype(
        jnp.float32
    )


def _stats_body(h_ref, ids_ref, s_ref, q_ref, n_ref):
    i = pl.program_id(0)
    h = h_ref[...]
    oh_t = _onehot_t(ids_ref)

    @pl.when(i == 0)
    def _():
        s_ref[...] = jnp.zeros_like(s_ref)
        q_ref[...] = jnp.zeros_like(q_ref)
        n_ref[...] = jnp.zeros_like(n_ref)

    s_ref[...] += jnp.dot(oh_t, h, preferred_element_type=jnp.float32)
    q_ref[...] += jnp.dot(oh_t, h * h, preferred_element_type=jnp.float32)
    n_ref[...] += jnp.sum(oh_t, axis=1, keepdims=True)


def _apply_body(h_ref, c_ref, ids_ref, w_ref, b2_ref, s_ref, q_ref, n_ref,
                o_ref):
    cnt = jnp.maximum(n_ref[...], 1.0)
    mean8 = s_ref[...] / cnt
    var8 = jnp.maximum(q_ref[...] / cnt - mean8 * mean8, 0.0)
    rstd8 = lax.rsqrt(var8 + _EPS)
    oh_t = _onehot_t(ids_ref)
    mr = lax.dot_general(oh_t, jnp.concatenate([mean8, rstd8], axis=1),
                         (((0,), (0,)), ((), ())),
                         preferred_element_type=jnp.float32)
    mean_r = mr[:, :_DH]
    rstd_r = mr[:, _DH:]
    h = h_ref[...]
    gb = lax.dot_general(c_ref[...], w_ref[...], (((1,), (1,)), ((), ())),
                         preferred_element_type=jnp.float32)
    gamma = gb[:, :_DH] + b2_ref[0:1, :]
    beta = gb[:, _DH:] + b2_ref[1:2, :]
    o_ref[...] = (h - mean_r) * rstd_r * gamma + beta + h


def _stats_call(h, ids_row):
    return pl.pallas_call(
        _stats_body,
        grid=(_NB,),
        in_specs=[
            pl.BlockSpec((_BB, _DH), lambda i: (i, 0)),
            pl.BlockSpec((1, 1, _BB), lambda i: (i, 0, 0)),
        ],
        out_specs=[pl.BlockSpec((_ND, _DH), lambda i: (0, 0))] * 3,
        out_shape=[jax.ShapeDtypeStruct((_ND, _DH), jnp.float32)] * 3,
    )(h, ids_row)


def _apply_call(h, csel, ids_row, W, b2, s, q, n):
    full = lambda shape: pl.BlockSpec(shape, lambda i: (0,) * len(shape))
    return pl.pallas_call(
        _apply_body,
        grid=(_NB,),
        in_specs=[
            pl.BlockSpec((_BB, _DH), lambda i: (i, 0)),
            pl.BlockSpec((_BB, _DC), lambda i: (i, 0)),
            pl.BlockSpec((1, 1, _BB), lambda i: (i, 0, 0)),
            full((2 * _DH, _DC)),
            full((2, _DH)),
            full((_ND, _DH)),
            full((_ND, _DH)),
            full((_ND, _DH)),
        ],
        out_specs=pl.BlockSpec((_BB, _DH), lambda i: (i, 0)),
        out_shape=jax.ShapeDtypeStruct((_B, _DH), jnp.float32),
    )(h, csel, ids_row, W, b2, s, q, n)


def kernel(h, cdir, domain_ids, W, b):
    ids = domain_ids.astype(jnp.int32)
    table = cdir.reshape(_ND * _B, _DC)
    ids_row = ids.reshape(_NB, 1, _BB)
    s, q, n = _stats_call(h, ids_row)
    ids_gate, _ = lax.optimization_barrier((ids, n))
    csel = _sc_gather_call()(ids_gate, table)
    return _apply_call(h, csel, ids_row, W, b.reshape(2, _DH), s, q, n)


# speedup vs baseline: 1.0592x; 1.0592x over previous
import functools

import jax
import jax.numpy as jnp
from jax import lax
from jax.experimental import pallas as pl
from jax.experimental.pallas import tpu as pltpu
from jax.experimental.pallas import tpu_sc as plsc

_B = 16384
_DH = 128
_DC = 128
_ND = 8
_EPS = 1e-05

_NW = 32
_RPW = _B // _NW
_CH = 128
_NCH = _RPW // _CH
_LANES = 16

_BB = 4096
_NB = _B // _BB


def _sc_gather_body(ids_hbm, table_hbm, out_hbm, ids_v, idx_v, rows_v, sem):
    wid = lax.axis_index("s") * 2 + lax.axis_index("c")
    base = wid * _RPW
    pltpu.sync_copy(ids_hbm.at[pl.ds(base, _RPW)], ids_v)

    for r in range(_NCH):

        def _mk_idx(j, carry, r=r):
            col = r * _CH + j * _LANES
            rows = base + col + lax.iota(jnp.int32, _LANES)
            idx_v[r, pl.ds(j * _LANES, _LANES)] = (
                ids_v[pl.ds(col, _LANES)] * _B + rows
            )
            return carry

        lax.fori_loop(0, _CH // _LANES, _mk_idx, 0)
    gathers = [
        pltpu.async_copy(
            table_hbm.at[idx_v.at[r]], rows_v.at[pl.ds(r * _CH, _CH)], sem
        )
        for r in range(_NCH)
    ]
    for cp in gathers:
        cp.wait()
    pltpu.sync_copy(rows_v, out_hbm.at[pl.ds(base, _RPW)])


@functools.lru_cache(maxsize=1)
def _sc_gather_call():
    mesh = plsc.VectorSubcoreMesh(core_axis_name="c", subcore_axis_name="s")
    return pl.kernel(
        _sc_gather_body,
        out_type=jax.ShapeDtypeStruct((_B, _DC), jnp.float32),
        mesh=mesh,
        scratch_types=[
            pltpu.VMEM((_RPW,), jnp.int32),
            pltpu.VMEM((_NCH, _CH), jnp.int32),
            pltpu.VMEM((_RPW, _DC), jnp.float32),
            pltpu.SemaphoreType.DMA,
        ],
    )


def _onehot_t(ids_ref):
    ids = jnp.broadcast_to(ids_ref[0], (_ND, _BB))
    return (ids == lax.broadcasted_iota(jnp.int32, (_ND, _BB), 0)).astype(
        jnp.float32
    )


def _stats_body(h_ref, ids_ref, s_ref, q_ref, n_ref):
    i = pl.program_id(0)
    h = h_ref[...]
    oh_t = _onehot_t(ids_ref)

    @pl.when(i == 0)
    def _():
        s_ref[...] = jnp.zeros_like(s_ref)
        q_ref[...] = jnp.zeros_like(q_ref)
        n_ref[...] = jnp.zeros_like(n_ref)

    s_ref[...] += jnp.dot(oh_t, h, preferred_element_type=jnp.float32)
    q_ref[...] += jnp.dot(oh_t, h * h, preferred_element_type=jnp.float32)
    n_ref[...] += jnp.sum(oh_t, axis=1, keepdims=True)


def _apply_body(h_ref, c_ref, ids_ref, w_ref, b2_ref, s_ref, q_ref, n_ref,
                o_ref):
    cnt = jnp.maximum(n_ref[...], 1.0)
    mean8 = s_ref[...] / cnt
    var8 = jnp.maximum(q_ref[...] / cnt - mean8 * mean8, 0.0)
    rstd8 = lax.rsqrt(var8 + _EPS)
    oh_t = _onehot_t(ids_ref)
    mr = lax.dot_general(oh_t, jnp.concatenate([mean8, rstd8], axis=1),
                         (((0,), (0,)), ((), ())),
                         preferred_element_type=jnp.float32)
    mean_r = mr[:, :_DH]
    rstd_r = mr[:, _DH:]
    h = h_ref[...]
    gb = lax.dot_general(c_ref[...], w_ref[...], (((1,), (1,)), ((), ())),
                         preferred_element_type=jnp.float32)
    gamma = gb[:, :_DH] + b2_ref[0:1, :]
    beta = gb[:, _DH:] + b2_ref[1:2, :]
    o_ref[...] = (h - mean_r) * rstd_r * gamma + beta + h


def _stats_call(h, ids_row):
    return pl.pallas_call(
        _stats_body,
        grid=(_NB,),
        in_specs=[
            pl.BlockSpec((_BB, _DH), lambda i: (i, 0)),
            pl.BlockSpec((1, 1, _BB), lambda i: (i, 0, 0)),
        ],
        out_specs=[pl.BlockSpec((_ND, _DH), lambda i: (0, 0))] * 3,
        out_shape=[jax.ShapeDtypeStruct((_ND, _DH), jnp.float32)] * 3,
    )(h, ids_row)


def _apply_call(h, csel, ids_row, W, b2, s, q, n):
    full = lambda shape: pl.BlockSpec(shape, lambda i: (0,) * len(shape))
    return pl.pallas_call(
        _apply_body,
        grid=(_NB,),
        in_specs=[
            pl.BlockSpec((_BB, _DH), lambda i: (i, 0)),
            pl.BlockSpec((_BB, _DC), lambda i: (i, 0)),
            pl.BlockSpec((1, 1, _BB), lambda i: (i, 0, 0)),
            full((2 * _DH, _DC)),
            full((2, _DH)),
            full((_ND, _DH)),
            full((_ND, _DH)),
            full((_ND, _DH)),
        ],
        out_specs=pl.BlockSpec((_BB, _DH), lambda i: (i, 0)),
        out_shape=jax.ShapeDtypeStruct((_B, _DH), jnp.float32),
    )(h, csel, ids_row, W, b2, s, q, n)


def kernel(h, cdir, domain_ids, W, b):
    ids = domain_ids.astype(jnp.int32)
    table = cdir.reshape(_ND * _B, _DC)
    csel = _sc_gather_call()(ids, table)
    ids_row = ids.reshape(_NB, 1, _BB)
    s, q, n = _stats_call(h, ids_row)
    return _apply_call(h, csel, ids_row, W, b.reshape(2, _DH), s, q, n)
